# baseline (speedup 1.0000x reference)
import functools

import jax
import jax.numpy as jnp
from jax import lax
from jax.experimental import pallas as pl
from jax.experimental.pallas import tpu as pltpu
from jax.experimental.pallas import tpu_sc as plsc

BLOCK_R = 256
NC, NS, L = 2, 16, 16
NW = NC * NS
CHUNK = 128


def _sc_gather(x_flat, t2, a_flat, R, N):
    b_per_w = R // NW
    CPR = b_per_w // CHUNK
    mesh = plsc.VectorSubcoreMesh(core_axis_name="c", subcore_axis_name="s")

    @functools.partial(
        pl.kernel,
        out_type=(
            jax.ShapeDtypeStruct((R // CHUNK, CHUNK), jnp.float32),
            jax.ShapeDtypeStruct((R // CHUNK, CHUNK), jnp.float32),
        ),
        mesh=mesh,
        scratch_types=[
            pltpu.VMEM((CPR, CHUNK), jnp.int32),
            pltpu.VMEM((CPR, CHUNK), jnp.int32),
            pltpu.VMEM((CPR, CHUNK), jnp.float32),
            pltpu.VMEM((CPR, CHUNK), jnp.float32),
            pltpu.SemaphoreType.DMA,
        ],
    )
    def gather_kernel(x_hbm, t_hbm, a_hbm, xt_hbm, at_hbm,
                      t_v, idx_v, xt_v, at_v, sem):
        wid = lax.axis_index("s") * NC + lax.axis_index("c")
        row0 = wid * CPR
        pltpu.sync_copy(t_hbm.at[pl.ds(row0, CPR)], t_v)
        for j in range(CPR):
            for k in range(CHUNK // L):
                tv = t_v[j, pl.ds(k * L, L)]
                flat0 = (wid * b_per_w + j * CHUNK + k * L) * N
                idx_v[j, pl.ds(k * L, L)] = (
                    flat0 + lax.iota(jnp.int32, L) * N + tv)
        for j in range(CPR):
            pltpu.async_copy(x_hbm.at[idx_v.at[j]], xt_v.at[j], sem).wait()
            pltpu.async_copy(a_hbm.at[t_v.at[j]], at_v.at[j], sem).wait()
        pltpu.sync_copy(xt_v, xt_hbm.at[pl.ds(row0, CPR)])
        pltpu.sync_copy(at_v, at_hbm.at[pl.ds(row0, CPR)])

    return gather_kernel(x_flat, t2, a_flat)


def kernel(inputs, targets, alpha):
    B, Q, N = inputs.shape
    R = B * Q
    x2 = inputs.reshape(R, N)
    x_flat = inputs.reshape(R * N)
    t2 = targets.reshape(R // CHUNK, CHUNK)
    a_flat = alpha.reshape(N)

    xt, at = _sc_gather(x_flat, t2, a_flat, R, N)

    def lse_body(x_ref, lse_ref):
        xb = x_ref[...]
        m = jnp.max(xb, axis=1, keepdims=True)
        s = jnp.sum(jnp.exp(xb - m), axis=1, keepdims=True)
        lse_ref[...] = (m + jnp.log(s)).reshape(1, BLOCK_R // CHUNK, CHUNK)

    lse = pl.pallas_call(
        lse_body,
        grid=(R // BLOCK_R,),
        in_specs=[pl.BlockSpec((BLOCK_R, N), lambda i: (i, 0))],
        out_specs=pl.BlockSpec((1, BLOCK_R // CHUNK, CHUNK),
                               lambda i: (i, 0, 0)),
        out_shape=jax.ShapeDtypeStruct(
            (R // BLOCK_R, BLOCK_R // CHUNK, CHUNK), jnp.float32),
    )(x2)
    lse = lse.reshape(R // CHUNK, CHUNK)

    def comb_body(xt_ref, at_ref, lse_ref, o_ref):
        logp = xt_ref[...] - lse_ref[...]
        p = jnp.exp(logp)
        q = 1.0 - p
        o_ref[0, 0] = jnp.sum(-at_ref[...] * q * q * logp) * (1.0 / R)

    out = pl.pallas_call(
        comb_body,
        out_specs=pl.BlockSpec(memory_space=pltpu.SMEM),
        out_shape=jax.ShapeDtypeStruct((1, 1), jnp.float32),
    )(xt, at, lse)
    return out[0, 0]


# speedup vs baseline: 2.2234x; 2.2234x over previous
import jax
import jax.numpy as jnp
from jax.experimental import pallas as pl
from jax.experimental.pallas import tpu as pltpu

BLOCK_R = 256


def kernel(inputs, targets, alpha):
    B, Q, N = inputs.shape
    R = B * Q
    x = inputs.reshape(R, N)
    t3 = targets.reshape(R // BLOCK_R, 1, BLOCK_R)

    def body(x_ref, t_ref, a_ref, out_ref):
        i = pl.program_id(0)
        xb = x_ref[...]
        t = t_ref[0, 0, :]
        m = jnp.max(xb, axis=1, keepdims=True)
        s = jnp.sum(jnp.exp(xb - m), axis=1, keepdims=True)
        ids = jax.lax.broadcasted_iota(jnp.int32, xb.shape, 1)
        mf = (ids == t[:, None]).astype(jnp.float32)
        ones = jnp.ones((N, 1), jnp.float32)
        xt = jax.lax.dot(mf * xb, ones,
                         preferred_element_type=jnp.float32)
        at = jax.lax.dot(mf, a_ref[...],
                         preferred_element_type=jnp.float32)
        logp = (xt - m) - jnp.log(s)
        p = jnp.exp(logp)
        q1 = 1.0 - p
        part = jnp.sum(-at * q1 * q1 * logp) * (1.0 / R)

        @pl.when(i == 0)
        def _():
            out_ref[0, 0] = 0.0

        out_ref[0, 0] += part

    out = pl.pallas_call(
        body,
        grid=(R // BLOCK_R,),
        in_specs=[
            pl.BlockSpec((BLOCK_R, N), lambda i: (i, 0)),
            pl.BlockSpec((1, 1, BLOCK_R), lambda i: (i, 0, 0)),
            pl.BlockSpec((N, 1), lambda i: (0, 0)),
        ],
        out_specs=pl.BlockSpec(memory_space=pltpu.SMEM),
        out_shape=jax.ShapeDtypeStruct((1, 1), jnp.float32),
    )(x, t3, alpha)
    return out[0, 0]
